# Initial kernel scaffold; baseline (speedup 1.0000x reference)
import jax
import jax.numpy as jnp
from jax.experimental import pallas as pl


def kernel(x_movie, x_user, edge_index_sims, edge_index_rev_rates, W1l, W1r, b1, W2l, W2r, b2, W3l, W3r, b3, Wlin1, blin1, Wlin2, blin2, Wlin3, blin3):
    raise NotImplementedError("write your pallas kernel here")


# speedup vs baseline: 3.8043x; 3.8043x over previous
import functools

import jax
import jax.numpy as jnp
from jax import lax
from jax.experimental import pallas as pl
from jax.experimental.pallas import tpu as pltpu
from jax.experimental.pallas import tpu_sc as plsc

N_NODE = 10000
D = 128
OUT = 64
D_AUG = 144
SPLIT = 72
W_HALF = 80
CNT = D - SPLIT
N_ACC = 10240
NC = 2
NS = 16
CHUNK = 128
NCH = 158
E_PAD = NS * NCH * CHUNK
ROWS_PER_TILE = N_ACC // NS


def _make_spmm():
    mesh = plsc.VectorSubcoreMesh(core_axis_name="c", subcore_axis_name="s",
                                  num_cores=NC, num_subcores=NS)

    @functools.partial(
        pl.kernel,
        out_type=jax.ShapeDtypeStruct((NC, N_ACC, W_HALF), jnp.float32),
        mesh=mesh,
        scratch_types=[
            pltpu.VMEM((NCH, CHUNK), jnp.int32),
            pltpu.VMEM((NCH, CHUNK), jnp.int32),
            pltpu.VMEM((CHUNK, W_HALF), jnp.float32),
            pltpu.VMEM_SHARED((N_ACC, W_HALF), jnp.float32),
        ],
        compiler_params=pltpu.CompilerParams(use_tc_tiling_on_sc=False),
    )
    def spmm(table, src_i, dst_i, zeros_h, out, idx_s, idx_d, rows, acc):
        c = lax.axis_index("c")
        s = lax.axis_index("s")
        base = s * ROWS_PER_TILE
        pltpu.sync_copy(zeros_h.at[pl.ds(base, ROWS_PER_TILE)],
                        acc.at[pl.ds(base, ROWS_PER_TILE)])
        pltpu.sync_copy(src_i.at[s], idx_s)
        pltpu.sync_copy(dst_i.at[s], idx_d)
        plsc.subcore_barrier()

        @pl.loop(0, NCH)
        def _(j):
            pltpu.sync_copy(table.at[c].at[idx_s.at[j]], rows)
            pltpu.sync_copy(rows, acc.at[idx_d.at[j]], add=True)

        plsc.subcore_barrier()
        pltpu.sync_copy(acc.at[pl.ds(base, ROWS_PER_TILE)],
                        out.at[c, pl.ds(base, ROWS_PER_TILE)])

    return spmm


_SPMM_CACHE = []


def _sc_aggregate(table2, src3, dst3, zeros_acc):
    if not _SPMM_CACHE:
        _SPMM_CACHE.append(_make_spmm())
    return _SPMM_CACHE[0](table2, src3, dst3, zeros_acc)


def _split_table(x_aug_cols):
    n = x_aug_cols.shape[0]
    zpad = jnp.zeros((n, W_HALF - SPLIT), jnp.float32)
    ones = jnp.ones((n, D_AUG - D), jnp.float32)
    h0 = jnp.concatenate([x_aug_cols[:, :SPLIT], zpad], axis=1)
    h1 = jnp.concatenate(
        [x_aug_cols[:, SPLIT:], ones,
         jnp.zeros((n, W_HALF - CNT - (D_AUG - D)), jnp.float32)], axis=1)
    return jnp.stack([h0, h1])


def _pad_edges(src, dst):
    pad = E_PAD - src.shape[0]
    src = jnp.concatenate([src, jnp.zeros((pad,), jnp.int32)])
    dst = jnp.concatenate([dst, jnp.full((pad,), N_NODE, jnp.int32)])
    return src.reshape(NS, NCH, CHUNK), dst.reshape(NS, NCH, CHUNK)


def _mm(a, b):
    return jnp.dot(a, b, preferred_element_type=jnp.float32,
                   precision=lax.Precision.HIGHEST)


def _mean_from_halves(h0, h1):
    feat = jnp.concatenate([h0[:, :SPLIT], h1[:, :CNT]], axis=1)
    cnt = h1[:, CNT:CNT + 1]
    return feat / jnp.maximum(cnt, 1.0)


def _dense1_body(a1h0_ref, a1h1_ref, a2h0_ref, a2h1_ref, xm_ref, xu_ref,
                 w1l_ref, w1r_ref, b1_ref, wl1_ref, bl1_ref,
                 w2l_ref, w2r_ref, b2_ref, wl2_ref, bl2_ref,
                 tb_ref, u2_ref):
    mean1 = _mean_from_halves(a1h0_ref[...], a1h1_ref[...])
    h0 = jax.nn.relu(_mm(mean1, w1l_ref[...]) + b1_ref[...]
                     + _mm(xm_ref[...], w1r_ref[...]))
    h = jax.nn.relu(_mm(h0, wl1_ref[...]) + bl1_ref[...])
    nrow = h.shape[0]
    zpad = jnp.zeros((nrow, W_HALF - SPLIT), jnp.float32)
    tb_ref[0] = jnp.concatenate([h[:, :SPLIT], zpad], axis=1)
    tb_ref[1] = jnp.concatenate(
        [h[:, SPLIT:], jnp.ones((nrow, D_AUG - D), jnp.float32),
         jnp.zeros((nrow, W_HALF - CNT - (D_AUG - D)), jnp.float32)], axis=1)
    mean2 = _mean_from_halves(a2h0_ref[...], a2h1_ref[...])
    u0 = jax.nn.relu(_mm(mean2, w2l_ref[...]) + b2_ref[...]
                     + _mm(xu_ref[...], w2r_ref[...]))
    u2_ref[...] = jax.nn.relu(_mm(u0, wl2_ref[...]) + bl2_ref[...])


def _dense2_body(a3h0_ref, a3h1_ref, u2_ref,
                 w3l_ref, w3r_ref, b3_ref, wl3_ref, bl3_ref, out_ref):
    mean3 = _mean_from_halves(a3h0_ref[...], a3h1_ref[...])
    u3 = jax.nn.relu(_mm(mean3, w3l_ref[...]) + b3_ref[...]
                     + _mm(u2_ref[...], w3r_ref[...]))
    out_ref[...] = _mm(u3, wl3_ref[...]) + bl3_ref[...]


_BLK = 400
_GRID = N_NODE // _BLK


def _row_spec(w):
    return pl.BlockSpec((_BLK, w), lambda i: (i, 0))


def _full_spec(h, w):
    return pl.BlockSpec((h, w), lambda i: (0, 0))


def kernel(x_movie, x_user, edge_index_sims, edge_index_rev_rates,
           W1l, W1r, b1, W2l, W2r, b2, W3l, W3r, b3,
           Wlin1, blin1, Wlin2, blin2, Wlin3, blin3):
    zeros_acc = jnp.zeros((N_ACC, W_HALF), jnp.float32)
    tA = _split_table(x_movie)

    srcS, dstS = _pad_edges(edge_index_sims[0], edge_index_sims[1])
    srcR, dstR = _pad_edges(edge_index_rev_rates[0], edge_index_rev_rates[1])

    agg1 = _sc_aggregate(tA, srcS, dstS, zeros_acc)
    agg2 = _sc_aggregate(tA, srcR, dstR, zeros_acc)

    b1r, b2r = b1.reshape(1, -1), b2.reshape(1, -1)
    bl1r, bl2r = blin1.reshape(1, -1), blin2.reshape(1, -1)
    tB, u2 = pl.pallas_call(
        _dense1_body,
        grid=(_GRID,),
        in_specs=[
            _row_spec(W_HALF), _row_spec(W_HALF),
            _row_spec(W_HALF), _row_spec(W_HALF),
            _row_spec(D), _row_spec(D),
            _full_spec(D, D), _full_spec(D, D), _full_spec(1, D),
            _full_spec(D, D), _full_spec(1, D),
            _full_spec(D, D), _full_spec(D, D), _full_spec(1, D),
            _full_spec(D, D), _full_spec(1, D),
        ],
        out_specs=[pl.BlockSpec((NC, _BLK, W_HALF), lambda i: (0, i, 0)),
                   _row_spec(D)],
        out_shape=[jax.ShapeDtypeStruct((NC, N_NODE, W_HALF), jnp.float32),
                   jax.ShapeDtypeStruct((N_NODE, D), jnp.float32)],
    )(agg1[0, :N_NODE], agg1[1, :N_NODE], agg2[0, :N_NODE], agg2[1, :N_NODE],
      x_movie, x_user,
      W1l, W1r, b1r, Wlin1, bl1r, W2l, W2r, b2r, Wlin2, bl2r)

    agg3 = _sc_aggregate(tB, srcR, dstR, zeros_acc)

    out = pl.pallas_call(
        _dense2_body,
        grid=(_GRID,),
        in_specs=[
            _row_spec(W_HALF), _row_spec(W_HALF), _row_spec(D),
            _full_spec(D, D), _full_spec(D, D), _full_spec(1, D),
            _full_spec(D, OUT), _full_spec(1, OUT),
        ],
        out_specs=_row_spec(OUT),
        out_shape=jax.ShapeDtypeStruct((N_NODE, OUT), jnp.float32),
    )(agg3[0, :N_NODE], agg3[1, :N_NODE], u2,
      W3l, W3r, b3.reshape(1, -1), Wlin3, blin3.reshape(1, -1))
    return out


# speedup vs baseline: 4.4421x; 1.1677x over previous
import functools

import jax
import jax.numpy as jnp
from jax import lax
from jax.experimental import pallas as pl
from jax.experimental.pallas import tpu as pltpu
from jax.experimental.pallas import tpu_sc as plsc

N_NODE = 10000
D = 128
OUT = 64
D_AUG = 144
SPLIT = 72
W_HALF = 80
CNT = D - SPLIT
N_ACC = 10240
NC = 2
NS = 16
CHUNK = 128
NCH = 158
E_PAD = NS * NCH * CHUNK
ROWS_PER_TILE = N_ACC // NS


def _make_spmm():
    mesh = plsc.VectorSubcoreMesh(core_axis_name="c", subcore_axis_name="s",
                                  num_cores=NC, num_subcores=NS)

    @functools.partial(
        pl.kernel,
        out_type=jax.ShapeDtypeStruct((NC, N_ACC, W_HALF), jnp.float32),
        mesh=mesh,
        scratch_types=[
            pltpu.VMEM((NCH, CHUNK), jnp.int32),
            pltpu.VMEM((NCH, CHUNK), jnp.int32),
            pltpu.VMEM((CHUNK, W_HALF), jnp.float32),
            pltpu.VMEM((CHUNK, W_HALF), jnp.float32),
            pltpu.VMEM_SHARED((N_ACC, W_HALF), jnp.float32),
            pltpu.SemaphoreType.DMA,
            pltpu.SemaphoreType.DMA,
        ],
        compiler_params=pltpu.CompilerParams(use_tc_tiling_on_sc=False),
    )
    def spmm(table, src_i, dst_i, zeros_h, out,
             idx_s, idx_d, rows0, rows1, acc, sem0, sem1):
        c = lax.axis_index("c")
        s = lax.axis_index("s")
        base = s * ROWS_PER_TILE
        pltpu.sync_copy(zeros_h.at[pl.ds(base, ROWS_PER_TILE)],
                        acc.at[pl.ds(base, ROWS_PER_TILE)])
        pltpu.sync_copy(src_i.at[s], idx_s)
        pltpu.sync_copy(dst_i.at[s], idx_d)
        plsc.subcore_barrier()

        bufs = (rows0, rows1)
        sems = (sem0, sem1)

        def gather(j, b):
            return pltpu.async_copy(table.at[c].at[idx_s.at[j]],
                                    bufs[b], sems[b])

        gather(0, 0)

        @pl.loop(0, NCH, step=2)
        def _(j0):
            for b in range(2):
                j = j0 + b
                pltpu.make_async_copy(table.at[c].at[idx_s.at[j]],
                                      bufs[b], sems[b]).wait()

                @pl.when(j + 1 < NCH)
                def _():
                    gather(j + 1, 1 - b)

                pltpu.sync_copy(bufs[b], acc.at[idx_d.at[j]], add=True)

        plsc.subcore_barrier()
        pltpu.sync_copy(acc.at[pl.ds(base, ROWS_PER_TILE)],
                        out.at[c, pl.ds(base, ROWS_PER_TILE)])

    return spmm


_SPMM_CACHE = []


def _sc_aggregate(table2, src3, dst3, zeros_acc):
    if not _SPMM_CACHE:
        _SPMM_CACHE.append(_make_spmm())
    return _SPMM_CACHE[0](table2, src3, dst3, zeros_acc)


def _split_table(x_aug_cols):
    n = x_aug_cols.shape[0]
    zpad = jnp.zeros((n, W_HALF - SPLIT), jnp.float32)
    ones = jnp.ones((n, D_AUG - D), jnp.float32)
    h0 = jnp.concatenate([x_aug_cols[:, :SPLIT], zpad], axis=1)
    h1 = jnp.concatenate(
        [x_aug_cols[:, SPLIT:], ones,
         jnp.zeros((n, W_HALF - CNT - (D_AUG - D)), jnp.float32)], axis=1)
    return jnp.stack([h0, h1])


def _pad_edges(src, dst):
    pad = E_PAD - src.shape[0]
    src = jnp.concatenate([src, jnp.zeros((pad,), jnp.int32)])
    dst = jnp.concatenate([dst, jnp.full((pad,), N_NODE, jnp.int32)])
    return src.reshape(NS, NCH, CHUNK), dst.reshape(NS, NCH, CHUNK)


def _mm(a, b):
    return jnp.dot(a, b, preferred_element_type=jnp.float32,
                   precision=lax.Precision.HIGHEST)


def _mean_from_halves(h0, h1):
    feat = jnp.concatenate([h0[:, :SPLIT], h1[:, :CNT]], axis=1)
    cnt = h1[:, CNT:CNT + 1]
    return feat / jnp.maximum(cnt, 1.0)


def _dense1_body(a1h0_ref, a1h1_ref, a2h0_ref, a2h1_ref, xm_ref, xu_ref,
                 w1l_ref, w1r_ref, b1_ref, wl1_ref, bl1_ref,
                 w2l_ref, w2r_ref, b2_ref, wl2_ref, bl2_ref,
                 tb_ref, u2_ref):
    mean1 = _mean_from_halves(a1h0_ref[...], a1h1_ref[...])
    h0 = jax.nn.relu(_mm(mean1, w1l_ref[...]) + b1_ref[...]
                     + _mm(xm_ref[...], w1r_ref[...]))
    h = jax.nn.relu(_mm(h0, wl1_ref[...]) + bl1_ref[...])
    nrow = h.shape[0]
    zpad = jnp.zeros((nrow, W_HALF - SPLIT), jnp.float32)
    tb_ref[0] = jnp.concatenate([h[:, :SPLIT], zpad], axis=1)
    tb_ref[1] = jnp.concatenate(
        [h[:, SPLIT:], jnp.ones((nrow, D_AUG - D), jnp.float32),
         jnp.zeros((nrow, W_HALF - CNT - (D_AUG - D)), jnp.float32)], axis=1)
    mean2 = _mean_from_halves(a2h0_ref[...], a2h1_ref[...])
    u0 = jax.nn.relu(_mm(mean2, w2l_ref[...]) + b2_ref[...]
                     + _mm(xu_ref[...], w2r_ref[...]))
    u2_ref[...] = jax.nn.relu(_mm(u0, wl2_ref[...]) + bl2_ref[...])


def _dense2_body(a3h0_ref, a3h1_ref, u2_ref,
                 w3l_ref, w3r_ref, b3_ref, wl3_ref, bl3_ref, out_ref):
    mean3 = _mean_from_halves(a3h0_ref[...], a3h1_ref[...])
    u3 = jax.nn.relu(_mm(mean3, w3l_ref[...]) + b3_ref[...]
                     + _mm(u2_ref[...], w3r_ref[...]))
    out_ref[...] = _mm(u3, wl3_ref[...]) + bl3_ref[...]


_BLK = 400
_GRID = N_NODE // _BLK


def _row_spec(w):
    return pl.BlockSpec((_BLK, w), lambda i: (i, 0))


def _full_spec(h, w):
    return pl.BlockSpec((h, w), lambda i: (0, 0))


def kernel(x_movie, x_user, edge_index_sims, edge_index_rev_rates,
           W1l, W1r, b1, W2l, W2r, b2, W3l, W3r, b3,
           Wlin1, blin1, Wlin2, blin2, Wlin3, blin3):
    zeros_acc = jnp.zeros((N_ACC, W_HALF), jnp.float32)
    tA = _split_table(x_movie)

    srcS, dstS = _pad_edges(edge_index_sims[0], edge_index_sims[1])
    srcR, dstR = _pad_edges(edge_index_rev_rates[0], edge_index_rev_rates[1])

    agg1 = _sc_aggregate(tA, srcS, dstS, zeros_acc)
    agg2 = _sc_aggregate(tA, srcR, dstR, zeros_acc)

    b1r, b2r = b1.reshape(1, -1), b2.reshape(1, -1)
    bl1r, bl2r = blin1.reshape(1, -1), blin2.reshape(1, -1)
    tB, u2 = pl.pallas_call(
        _dense1_body,
        grid=(_GRID,),
        in_specs=[
            _row_spec(W_HALF), _row_spec(W_HALF),
            _row_spec(W_HALF), _row_spec(W_HALF),
            _row_spec(D), _row_spec(D),
            _full_spec(D, D), _full_spec(D, D), _full_spec(1, D),
            _full_spec(D, D), _full_spec(1, D),
            _full_spec(D, D), _full_spec(D, D), _full_spec(1, D),
            _full_spec(D, D), _full_spec(1, D),
        ],
        out_specs=[pl.BlockSpec((NC, _BLK, W_HALF), lambda i: (0, i, 0)),
                   _row_spec(D)],
        out_shape=[jax.ShapeDtypeStruct((NC, N_NODE, W_HALF), jnp.float32),
                   jax.ShapeDtypeStruct((N_NODE, D), jnp.float32)],
    )(agg1[0, :N_NODE], agg1[1, :N_NODE], agg2[0, :N_NODE], agg2[1, :N_NODE],
      x_movie, x_user,
      W1l, W1r, b1r, Wlin1, bl1r, W2l, W2r, b2r, Wlin2, bl2r)

    agg3 = _sc_aggregate(tB, srcR, dstR, zeros_acc)

    out = pl.pallas_call(
        _dense2_body,
        grid=(_GRID,),
        in_specs=[
            _row_spec(W_HALF), _row_spec(W_HALF), _row_spec(D),
            _full_spec(D, D), _full_spec(D, D), _full_spec(1, D),
            _full_spec(D, OUT), _full_spec(1, OUT),
        ],
        out_specs=_row_spec(OUT),
        out_shape=jax.ShapeDtypeStruct((N_NODE, OUT), jnp.float32),
    )(agg3[0, :N_NODE], agg3[1, :N_NODE], u2,
      W3l, W3r, b3.reshape(1, -1), Wlin3, blin3.reshape(1, -1))
    return out
